# Initial kernel scaffold; baseline (speedup 1.0000x reference)
import jax
import jax.numpy as jnp
from jax.experimental import pallas as pl


def kernel(v, e, v0, e0, alpha, beta, vidx, eidx, W_node, W_edge):
    raise NotImplementedError("write your pallas kernel here")


# speedup vs baseline: 1.4602x; 1.4602x over previous
import functools

import jax
import jax.numpy as jnp
from jax import lax
from jax.experimental import pallas as pl
from jax.experimental.pallas import tpu as pltpu
from jax.experimental.pallas import tpu_sc as plsc

NC = 2
NS = 16
L = 16
NW = NC * NS
DH = 128

f32 = jnp.float32


def _sc_mesh():
    return plsc.VectorSubcoreMesh(
        core_axis_name="c", subcore_axis_name="s", num_cores=NC, num_subcores=NS
    )


def _splits(total):
    main = (total // NS) // 8 * 8
    return main, total - main * NS


def _chunk_counts(E, ch):
    assert E % ch == 0
    nchunks = E // ch
    return nchunks // NW, nchunks % NW


def _zero_rows(zsrc, dst, s, row_main, row_tail):
    r0 = s * row_main
    pltpu.sync_copy(zsrc.at[pl.ds(r0, row_main)], dst.at[pl.ds(r0, row_main)])
    if row_tail:
        @pl.when(s == NS - 1)
        def _():
            b = row_main * NS
            pltpu.sync_copy(zsrc.at[pl.ds(b, row_tail)], dst.at[pl.ds(b, row_tail)])


def _dump_rows(src, dst, s, row_main, row_tail):
    r0 = s * row_main
    pltpu.sync_copy(src.at[pl.ds(r0, row_main)], dst.at[pl.ds(r0, row_main)])
    if row_tail:
        @pl.when(s == NS - 1)
        def _():
            b = row_main * NS
            pltpu.sync_copy(src.at[pl.ds(b, row_tail)], dst.at[pl.ds(b, row_tail)])


def _build_phase_a(N, M, E, ch):
    ntbase, ntrem = _chunk_counts(E, ch)
    rm, rm_tail = _splits(M)
    rn, rn_tail = _splits(N)

    @functools.partial(
        pl.kernel,
        mesh=_sc_mesh(),
        out_type=[
            jax.ShapeDtypeStruct((NC, M, DH), f32),
            jax.ShapeDtypeStruct((NC, M, DH), f32),
            jax.ShapeDtypeStruct((NC, M, DH), f32),
            jax.ShapeDtypeStruct((NC, N, DH), f32),
        ],
        scratch_types=[
            pltpu.VMEM((ch,), jnp.int32),
            pltpu.VMEM((ch,), jnp.int32),
            pltpu.VMEM((ch, DH), f32),
            pltpu.VMEM((ch, DH), f32),
            pltpu.VMEM_SHARED((N, DH), f32),
            pltpu.SemaphoreType.DMA,
        ],
    )
    def phase_a(vlo_h, vhi_h, vidx_h, eidx_h, zn_h, ones_h,
                eslo_o, eshi_o, cnte_o, cntv_o,
                vi_v, ei_v, rows_v, ones_v, acc, sem):
        c = lax.axis_index("c")
        s = lax.axis_index("s")
        wid = s * NC + c
        nt = ntbase + jnp.where(wid < ntrem, 1, 0)
        pltpu.sync_copy(ones_h, ones_v)

        def stage(table_h, out_ref, rows_main, rows_tail, by_vidx, gather):
            _zero_rows(zn_h, acc, s, rn, rn_tail)
            plsc.subcore_barrier()

            def body(t, carry):
                off = pl.multiple_of((wid + t * NW) * ch, 128)
                if gather or not by_vidx:
                    pltpu.sync_copy(eidx_h.at[pl.ds(off, ch)], ei_v)
                if gather or by_vidx:
                    pltpu.sync_copy(vidx_h.at[pl.ds(off, ch)], vi_v)
                if gather:
                    pltpu.async_copy(table_h.at[vi_v], rows_v, sem).wait()
                    pltpu.sync_copy(rows_v, acc.at[ei_v], add=True)
                elif by_vidx:
                    pltpu.sync_copy(ones_v, acc.at[vi_v], add=True)
                else:
                    pltpu.sync_copy(ones_v, acc.at[ei_v], add=True)
                return carry

            lax.fori_loop(0, nt, body, 0)
            plsc.subcore_barrier()
            _dump_rows(acc, out_ref.at[c], s, rows_main, rows_tail)
            plsc.subcore_barrier()

        stage(vlo_h, eslo_o, rm, rm_tail, by_vidx=False, gather=True)
        stage(vhi_h, eshi_o, rm, rm_tail, by_vidx=False, gather=True)
        stage(None, cnte_o, rm, rm_tail, by_vidx=False, gather=False)
        stage(None, cntv_o, rn, rn_tail, by_vidx=True, gather=False)

    return phase_a


def _build_phase_c(N, M, E, D, ch):
    DX = D + DH
    ntbase, ntrem = _chunk_counts(E, ch)
    rn, rn_tail = _splits(N)
    KD = D // L
    KH = DH // L

    @functools.partial(
        pl.kernel,
        mesh=_sc_mesh(),
        out_type=[
            jax.ShapeDtypeStruct((NC, N, DH), f32),
            jax.ShapeDtypeStruct((NC, N, DH), f32),
            jax.ShapeDtypeStruct((E, L), f32),
        ],
        scratch_types=[
            pltpu.VMEM((ch,), jnp.int32),
            pltpu.VMEM((ch,), jnp.int32),
            pltpu.VMEM((ch, D), f32),
            pltpu.VMEM((ch, DX), f32),
            pltpu.VMEM((ch, DH), f32),
            pltpu.VMEM((ch, L), f32),
            pltpu.VMEM((ch, 2 * L), f32),
            pltpu.VMEM_SHARED((N, DH), f32),
            pltpu.SemaphoreType.DMA,
            pltpu.SemaphoreType.DMA,
        ],
    )
    def phase_c(vhat_h, ex_h, ehi_h, vidx_h, eidx_h, zn_h,
                nslo_o, nshi_o, sims_o,
                vi_v, ei_v, vrows, erows, half_v, sims_v, tmp_v,
                acc, sem, sem2):
        c = lax.axis_index("c")
        s = lax.axis_index("s")
        wid = s * NC + c
        nt = ntbase + jnp.where(wid < ntrem, 1, 0)

        _zero_rows(zn_h, acc, s, rn, rn_tail)
        plsc.subcore_barrier()

        def body1(t, carry):
            off = pl.multiple_of((wid + t * NW) * ch, 128)
            pltpu.sync_copy(vidx_h.at[pl.ds(off, ch)], vi_v)
            pltpu.sync_copy(eidx_h.at[pl.ds(off, ch)], ei_v)
            cp1 = pltpu.async_copy(vhat_h.at[vi_v], vrows, sem)
            cp2 = pltpu.async_copy(ex_h.at[ei_v], erows, sem2)
            cp1.wait()
            cp2.wait()

            def ibody(i, carry2):
                accs = [
                    vrows[i, pl.ds(k * L, L)] * erows[i, pl.ds(k * L, L)]
                    for k in range(4)
                ]
                for k in range(4, KD):
                    accs[k % 4] = accs[k % 4] + (
                        vrows[i, pl.ds(k * L, L)] * erows[i, pl.ds(k * L, L)]
                    )
                d = (accs[0] + accs[1]) + (accs[2] + accs[3])
                for sh in (8, 4, 2, 1):
                    tmp_v[i, pl.ds(0, L)] = d
                    tmp_v[i, pl.ds(L, L)] = d
                    d = d + tmp_v[i, pl.ds(sh, L)]
                sw = erows[i, pl.ds(D, L)] / (1.0 + jnp.exp(-d))
                sims_v[i] = sw
                for k in range(KH):
                    half_v[i, pl.ds(k * L, L)] = sw * erows[i, pl.ds(k * L, L)]
                return carry2

            lax.fori_loop(0, ch, ibody, 0)
            pltpu.sync_copy(half_v, acc.at[vi_v], add=True)
            pltpu.sync_copy(sims_v, sims_o.at[pl.ds(off, ch)])
            return carry

        lax.fori_loop(0, nt, body1, 0)
        plsc.subcore_barrier()
        _dump_rows(acc, nslo_o.at[c], s, rn, rn_tail)
        plsc.subcore_barrier()

        _zero_rows(zn_h, acc, s, rn, rn_tail)
        plsc.subcore_barrier()

        def body2(t, carry):
            off = pl.multiple_of((wid + t * NW) * ch, 128)
            pltpu.sync_copy(vidx_h.at[pl.ds(off, ch)], vi_v)
            pltpu.sync_copy(sims_o.at[pl.ds(off, ch)], sims_v)
            pltpu.sync_copy(eidx_h.at[pl.ds(off, ch)], ei_v)
            pltpu.async_copy(ehi_h.at[ei_v], half_v, sem).wait()

            def ibody(i, carry2):
                sw = sims_v[i]
                for k in range(KH):
                    half_v[i, pl.ds(k * L, L)] = sw * half_v[i, pl.ds(k * L, L)]
                return carry2

            lax.fori_loop(0, ch, ibody, 0)
            pltpu.sync_copy(half_v, acc.at[vi_v], add=True)
            return carry

        lax.fori_loop(0, nt, body2, 0)
        plsc.subcore_barrier()
        _dump_rows(acc, nshi_o.at[c], s, rn, rn_tail)

    return phase_c


def _vhat(x, bn):
    n, d = x.shape

    def body(x_ref, o_ref):
        xb = x_ref[...]
        ss = jnp.sum(xb * xb, axis=1, keepdims=True)
        o_ref[...] = xb / jnp.maximum(jnp.sqrt(ss), 1e-8)

    return pl.pallas_call(
        body,
        grid=(n // bn,),
        in_specs=[pl.BlockSpec((bn, d), lambda i: (i, 0))],
        out_specs=pl.BlockSpec((bn, d), lambda i: (i, 0)),
        out_shape=jax.ShapeDtypeStruct((n, d), f32),
    )(x)


def _edge_dense(alpha2, beta2, eslo, eshi, cnte, e0, W_edge, bm):
    M, D = e0.shape
    DX = D + DH

    def body(a_ref, b_ref, lo_ref, hi_ref, ce_ref, e0_ref, w_ref,
             edge_ref, ex_ref):
        a = a_ref[0, 0]
        bt = b_ref[0, 0]
        es = jnp.concatenate(
            [lo_ref[0] + lo_ref[1], hi_ref[0] + hi_ref[1]], axis=1
        )
        cnt = jnp.maximum(ce_ref[0, :, 0:1] + ce_ref[1, :, 0:1], 1.0)
        mix = (1.0 - a) * (es / cnt) + a * e0_ref[...]
        mm = lax.dot_general(mix, w_ref[...], (((1,), (1,)), ((), ())),
                             preferred_element_type=f32)
        ed = jnp.maximum(bt * mm + (1.0 - bt) * mix, 0.0)
        edge_ref[...] = ed
        ss = jnp.sum(ed * ed, axis=1, keepdims=True)
        ne = jnp.maximum(jnp.sqrt(ss), 1e-8)
        ex_ref[...] = jnp.concatenate(
            [ed / ne,
             jnp.broadcast_to(ne, (bm, L)),
             jnp.zeros((bm, DH - L), f32)],
            axis=1,
        )

    return pl.pallas_call(
        body,
        grid=(M // bm,),
        in_specs=[
            pl.BlockSpec((1, 1), lambda i: (0, 0)),
            pl.BlockSpec((1, 1), lambda i: (0, 0)),
            pl.BlockSpec((NC, bm, DH), lambda i: (0, i, 0)),
            pl.BlockSpec((NC, bm, DH), lambda i: (0, i, 0)),
            pl.BlockSpec((NC, bm, DH), lambda i: (0, i, 0)),
            pl.BlockSpec((bm, D), lambda i: (i, 0)),
            pl.BlockSpec((D, D), lambda i: (0, 0)),
        ],
        out_specs=[
            pl.BlockSpec((bm, D), lambda i: (i, 0)),
            pl.BlockSpec((bm, DX), lambda i: (i, 0)),
        ],
        out_shape=[
            jax.ShapeDtypeStruct((M, D), f32),
            jax.ShapeDtypeStruct((M, DX), f32),
        ],
    )(alpha2, beta2, eslo, eshi, cnte, e0, W_edge)


def _node_dense(alpha2, beta2, nslo, nshi, cntv, v0, W_node, bn):
    N, D = v0.shape

    def body(a_ref, b_ref, lo_ref, hi_ref, cv_ref, v0_ref, w_ref, node_ref):
        a = a_ref[0, 0]
        bt = b_ref[0, 0]
        ns = jnp.concatenate(
            [lo_ref[0] + lo_ref[1], hi_ref[0] + hi_ref[1]], axis=1
        )
        cnt = jnp.maximum(cv_ref[0, :, 0:1] + cv_ref[1, :, 0:1], 1.0)
        mix = (1.0 - a) * (ns / cnt) + a * v0_ref[...]
        mm = lax.dot_general(mix, w_ref[...], (((1,), (1,)), ((), ())),
                             preferred_element_type=f32)
        node_ref[...] = bt * mm + (1.0 - bt) * mix

    return pl.pallas_call(
        body,
        grid=(N // bn,),
        in_specs=[
            pl.BlockSpec((1, 1), lambda i: (0, 0)),
            pl.BlockSpec((1, 1), lambda i: (0, 0)),
            pl.BlockSpec((NC, bn, DH), lambda i: (0, i, 0)),
            pl.BlockSpec((NC, bn, DH), lambda i: (0, i, 0)),
            pl.BlockSpec((NC, bn, DH), lambda i: (0, i, 0)),
            pl.BlockSpec((bn, D), lambda i: (i, 0)),
            pl.BlockSpec((D, D), lambda i: (0, 0)),
        ],
        out_specs=pl.BlockSpec((bn, D), lambda i: (i, 0)),
        out_shape=jax.ShapeDtypeStruct((N, D), f32),
    )(alpha2, beta2, nslo, nshi, cntv, v0, W_node)


def kernel(v, e, v0, e0, alpha, beta, vidx, eidx, W_node, W_edge):
    N, D = v.shape
    M = e0.shape[0]
    E = vidx.shape[0]
    vidx = vidx.astype(jnp.int32)
    eidx = eidx.astype(jnp.int32)
    alpha2 = jnp.reshape(alpha, (1, 1)).astype(f32)
    beta2 = jnp.reshape(beta, (1, 1)).astype(f32)

    ch_a, ch_c = 128, 32
    zn = jnp.zeros((N, DH), f32)

    eslo, eshi, cnte, cntv = _build_phase_a(N, M, E, ch_a)(
        v[:, :DH], v[:, DH:], vidx, eidx, zn, jnp.ones((ch_a, DH), f32),
    )

    vhat = _vhat(v, 1000)
    edge, ex = _edge_dense(alpha2, beta2, eslo, eshi, cnte, e0, W_edge, 200)
    ehi = ex[:, DH:D]

    nslo, nshi, _ = _build_phase_c(N, M, E, D, ch_c)(
        vhat, ex, ehi, vidx, eidx, zn,
    )

    node = _node_dense(alpha2, beta2, nslo, nshi, cntv, v0, W_node, 200)
    return (node, edge)


# speedup vs baseline: 2.1172x; 1.4499x over previous
import functools

import jax
import jax.numpy as jnp
from jax import lax
from jax.experimental import pallas as pl
from jax.experimental.pallas import tpu as pltpu
from jax.experimental.pallas import tpu_sc as plsc

NC = 2
NS = 16
L = 16
NW = NC * NS
DH = 128

f32 = jnp.float32


def _sc_mesh():
    return plsc.VectorSubcoreMesh(
        core_axis_name="c", subcore_axis_name="s", num_cores=NC, num_subcores=NS
    )


def _splits(total):
    main = (total // NS) // 8 * 8
    return main, total - main * NS


def _chunk_counts(E, ch):
    assert E % ch == 0
    nchunks = E // ch
    return nchunks // NW, nchunks % NW


def _zero_rows(zsrc, dst, s, row_main, row_tail):
    r0 = s * row_main
    pltpu.sync_copy(zsrc.at[pl.ds(r0, row_main)], dst.at[pl.ds(r0, row_main)])
    if row_tail:
        @pl.when(s == NS - 1)
        def _():
            b = row_main * NS
            pltpu.sync_copy(zsrc.at[pl.ds(b, row_tail)], dst.at[pl.ds(b, row_tail)])


def _dump_rows(src, dst, s, row_main, row_tail):
    r0 = s * row_main
    pltpu.sync_copy(src.at[pl.ds(r0, row_main)], dst.at[pl.ds(r0, row_main)])
    if row_tail:
        @pl.when(s == NS - 1)
        def _():
            b = row_main * NS
            pltpu.sync_copy(src.at[pl.ds(b, row_tail)], dst.at[pl.ds(b, row_tail)])


def _pipelined_chunks(nt, ntmax, issue, work):
    issue(0, 0)
    issue(1, 1)

    def pair(p, carry):
        for sl in (0, 1):
            t = 2 * p + sl

            @pl.when(t < nt)
            def _():
                work(t, sl)

            @pl.when(t + 2 < nt)
            def _():
                issue(t + 2, sl)
        return carry

    lax.fori_loop(0, (ntmax + 1) // 2, pair, 0)


def _build_phase_a(N, M, E, ch):
    ntbase, ntrem = _chunk_counts(E, ch)
    ntmax = ntbase + (1 if ntrem else 0)
    rm, rm_tail = _splits(M)
    rn, rn_tail = _splits(N)

    @functools.partial(
        pl.kernel,
        mesh=_sc_mesh(),
        out_type=[
            jax.ShapeDtypeStruct((NC, M, DH), f32),
            jax.ShapeDtypeStruct((NC, M, DH), f32),
            jax.ShapeDtypeStruct((NC, M, DH), f32),
            jax.ShapeDtypeStruct((NC, N, DH), f32),
        ],
        scratch_types=[
            pltpu.VMEM((ch,), jnp.int32),
            pltpu.VMEM((ch,), jnp.int32),
            pltpu.VMEM((ch,), jnp.int32),
            pltpu.VMEM((ch,), jnp.int32),
            pltpu.VMEM((ch, DH), f32),
            pltpu.VMEM((ch, DH), f32),
            pltpu.VMEM((ch, DH), f32),
            pltpu.VMEM_SHARED((N, DH), f32),
            pltpu.SemaphoreType.DMA,
            pltpu.SemaphoreType.DMA,
        ],
    )
    def phase_a(vlo_h, vhi_h, vidx_h, eidx_h, zn_h, ones_h,
                eslo_o, eshi_o, cnte_o, cntv_o,
                vi0, vi1, ei0, ei1, rows0, rows1, ones_v, acc, sem0, sem1):
        c = lax.axis_index("c")
        s = lax.axis_index("s")
        wid = s * NC + c
        nt = ntbase + jnp.where(wid < ntrem, 1, 0)
        pltpu.sync_copy(ones_h, ones_v)
        vi = (vi0, vi1)
        ei = (ei0, ei1)
        rows = (rows0, rows1)
        sems = (sem0, sem1)

        def gather_stage(table_h, out_ref, rows_main, rows_tail):
            _zero_rows(zn_h, acc, s, rn, rn_tail)
            plsc.subcore_barrier()

            def issue(t, sl):
                off = pl.multiple_of((wid + t * NW) * ch, 8)
                pltpu.sync_copy(vidx_h.at[pl.ds(off, ch)], vi[sl])
                pltpu.sync_copy(eidx_h.at[pl.ds(off, ch)], ei[sl])
                pltpu.async_copy(table_h.at[vi[sl]], rows[sl], sems[sl])

            def work(t, sl):
                pltpu.make_async_copy(table_h.at[vi[sl]], rows[sl], sems[sl]).wait()
                pltpu.sync_copy(rows[sl], acc.at[ei[sl]], add=True)

            _pipelined_chunks(nt, ntmax, issue, work)
            plsc.subcore_barrier()
            _dump_rows(acc, out_ref.at[c], s, rows_main, rows_tail)
            plsc.subcore_barrier()

        def count_stage(out_ref, rows_main, rows_tail, by_vidx):
            _zero_rows(zn_h, acc, s, rn, rn_tail)
            plsc.subcore_barrier()
            idx_h = vidx_h if by_vidx else eidx_h

            def body(t, carry):
                off = pl.multiple_of((wid + t * NW) * ch, 8)
                pltpu.sync_copy(idx_h.at[pl.ds(off, ch)], vi0)
                pltpu.sync_copy(ones_v, acc.at[vi0], add=True)
                return carry

            lax.fori_loop(0, nt, body, 0)
            plsc.subcore_barrier()
            _dump_rows(acc, out_ref.at[c], s, rows_main, rows_tail)
            plsc.subcore_barrier()

        gather_stage(vlo_h, eslo_o, rm, rm_tail)
        gather_stage(vhi_h, eshi_o, rm, rm_tail)
        count_stage(cnte_o, rm, rm_tail, by_vidx=False)
        count_stage(cntv_o, rn, rn_tail, by_vidx=True)

    return phase_a


def _build_sims(N, M, E, D, ch):
    DX = D + DH
    ntbase, ntrem = _chunk_counts(E, ch)
    ntmax = ntbase + (1 if ntrem else 0)
    KD = D // L

    @functools.partial(
        pl.kernel,
        mesh=_sc_mesh(),
        out_type=[jax.ShapeDtypeStruct((E, L), f32)],
        scratch_types=[
            pltpu.VMEM((ch,), jnp.int32),
            pltpu.VMEM((ch,), jnp.int32),
            pltpu.VMEM((ch,), jnp.int32),
            pltpu.VMEM((ch,), jnp.int32),
            pltpu.VMEM((ch, D), f32),
            pltpu.VMEM((ch, D), f32),
            pltpu.VMEM((ch, DX), f32),
            pltpu.VMEM((ch, DX), f32),
            pltpu.VMEM((ch, L), f32),
            pltpu.VMEM((ch, 2 * L), f32),
            pltpu.SemaphoreType.DMA,
            pltpu.SemaphoreType.DMA,
            pltpu.SemaphoreType.DMA,
            pltpu.SemaphoreType.DMA,
        ],
    )
    def sims_k(vhat_h, ex_h, vidx_h, eidx_h,
               sims_o,
               vi0, vi1, ei0, ei1, vrows0, vrows1, erows0, erows1,
               sims_v, tmp_v, semv0, semv1, seme0, seme1):
        c = lax.axis_index("c")
        s = lax.axis_index("s")
        wid = s * NC + c
        nt = ntbase + jnp.where(wid < ntrem, 1, 0)
        vi = (vi0, vi1)
        ei = (ei0, ei1)
        vrows = (vrows0, vrows1)
        erows = (erows0, erows1)
        semv = (semv0, semv1)
        seme = (seme0, seme1)

        def issue(t, sl):
            off = pl.multiple_of((wid + t * NW) * ch, 8)
            pltpu.sync_copy(vidx_h.at[pl.ds(off, ch)], vi[sl])
            pltpu.sync_copy(eidx_h.at[pl.ds(off, ch)], ei[sl])
            pltpu.async_copy(vhat_h.at[vi[sl]], vrows[sl], semv[sl])
            pltpu.async_copy(ex_h.at[ei[sl]], erows[sl], seme[sl])

        def work(t, sl):
            off = pl.multiple_of((wid + t * NW) * ch, 8)
            pltpu.make_async_copy(vhat_h.at[vi[sl]], vrows[sl], semv[sl]).wait()
            pltpu.make_async_copy(ex_h.at[ei[sl]], erows[sl], seme[sl]).wait()
            vr = vrows[sl]
            er = erows[sl]

            def one(i):
                accs = [
                    vr[i, pl.ds(k * L, L)] * er[i, pl.ds(k * L, L)]
                    for k in range(4)
                ]
                for k in range(4, KD):
                    accs[k % 4] = accs[k % 4] + (
                        vr[i, pl.ds(k * L, L)] * er[i, pl.ds(k * L, L)]
                    )
                d = (accs[0] + accs[1]) + (accs[2] + accs[3])
                for sh in (8, 4, 2, 1):
                    tmp_v[i, pl.ds(0, L)] = d
                    tmp_v[i, pl.ds(L, L)] = d
                    d = d + tmp_v[i, pl.ds(sh, L)]
                sims_v[i] = er[i, pl.ds(D, L)] / (1.0 + jnp.exp(-d))

            def ibody(i2, carry2):
                one(2 * i2)
                one(2 * i2 + 1)
                return carry2

            lax.fori_loop(0, ch // 2, ibody, 0)
            pltpu.sync_copy(sims_v, sims_o.at[pl.ds(off, ch)])

        _pipelined_chunks(nt, ntmax, issue, work)

    return sims_k


def _build_scatter_halves(N, M, E, ch):
    ntbase, ntrem = _chunk_counts(E, ch)
    ntmax = ntbase + (1 if ntrem else 0)
    rn, rn_tail = _splits(N)
    KH = DH // L

    @functools.partial(
        pl.kernel,
        mesh=_sc_mesh(),
        out_type=[
            jax.ShapeDtypeStruct((NC, N, DH), f32),
            jax.ShapeDtypeStruct((NC, N, DH), f32),
        ],
        scratch_types=[
            pltpu.VMEM((ch,), jnp.int32),
            pltpu.VMEM((ch,), jnp.int32),
            pltpu.VMEM((ch,), jnp.int32),
            pltpu.VMEM((ch,), jnp.int32),
            pltpu.VMEM((ch, DH), f32),
            pltpu.VMEM((ch, DH), f32),
            pltpu.VMEM((ch, L), f32),
            pltpu.VMEM((ch, L), f32),
            pltpu.VMEM_SHARED((N, DH), f32),
            pltpu.SemaphoreType.DMA,
            pltpu.SemaphoreType.DMA,
        ],
    )
    def scatter_k(el_h, eh_h, sims_h, vidx_h, eidx_h, zn_h,
                  nslo_o, nshi_o,
                  vi0, vi1, ei0, ei1, rows0, rows1, sims0, sims1,
                  acc, sem0, sem1):
        c = lax.axis_index("c")
        s = lax.axis_index("s")
        wid = s * NC + c
        nt = ntbase + jnp.where(wid < ntrem, 1, 0)
        vi = (vi0, vi1)
        ei = (ei0, ei1)
        rows = (rows0, rows1)
        sims = (sims0, sims1)
        sems = (sem0, sem1)

        def stage(table_h, out_ref):
            _zero_rows(zn_h, acc, s, rn, rn_tail)
            plsc.subcore_barrier()

            def issue(t, sl):
                off = pl.multiple_of((wid + t * NW) * ch, 8)
                pltpu.sync_copy(vidx_h.at[pl.ds(off, ch)], vi[sl])
                pltpu.sync_copy(eidx_h.at[pl.ds(off, ch)], ei[sl])
                pltpu.sync_copy(sims_h.at[pl.ds(off, ch)], sims[sl])
                pltpu.async_copy(table_h.at[ei[sl]], rows[sl], sems[sl])

            def work(t, sl):
                pltpu.make_async_copy(table_h.at[ei[sl]], rows[sl], sems[sl]).wait()
                rr = rows[sl]
                sv = sims[sl]

                def one(i):
                    sw = sv[i]
                    for k in range(KH):
                        rr[i, pl.ds(k * L, L)] = sw * rr[i, pl.ds(k * L, L)]

                def ibody(i2, carry2):
                    one(2 * i2)
                    one(2 * i2 + 1)
                    return carry2

                lax.fori_loop(0, ch // 2, ibody, 0)
                pltpu.sync_copy(rr, acc.at[vi[sl]], add=True)

            _pipelined_chunks(nt, ntmax, issue, work)
            plsc.subcore_barrier()
            _dump_rows(acc, out_ref.at[c], s, rn, rn_tail)
            plsc.subcore_barrier()

        stage(el_h, nslo_o)
        stage(eh_h, nshi_o)

    return scatter_k


def _vhat(x, bn):
    n, d = x.shape

    def body(x_ref, o_ref):
        xb = x_ref[...]
        ss = jnp.sum(xb * xb, axis=1, keepdims=True)
        o_ref[...] = xb / jnp.maximum(jnp.sqrt(ss), 1e-8)

    return pl.pallas_call(
        body,
        grid=(n // bn,),
        in_specs=[pl.BlockSpec((bn, d), lambda i: (i, 0))],
        out_specs=pl.BlockSpec((bn, d), lambda i: (i, 0)),
        out_shape=jax.ShapeDtypeStruct((n, d), f32),
    )(x)


def _edge_dense(alpha2, beta2, eslo, eshi, cnte, e0, W_edge, bm):
    M, D = e0.shape
    DX = D + DH

    def body(a_ref, b_ref, lo_ref, hi_ref, ce_ref, e0_ref, w_ref,
             edge_ref, ex_ref):
        a = a_ref[0, 0]
        bt = b_ref[0, 0]
        es = jnp.concatenate(
            [lo_ref[0] + lo_ref[1], hi_ref[0] + hi_ref[1]], axis=1
        )
        cnt = jnp.maximum(ce_ref[0, :, 0:1] + ce_ref[1, :, 0:1], 1.0)
        mix = (1.0 - a) * (es / cnt) + a * e0_ref[...]
        mm = lax.dot_general(mix, w_ref[...], (((1,), (1,)), ((), ())),
                             preferred_element_type=f32)
        ed = jnp.maximum(bt * mm + (1.0 - bt) * mix, 0.0)
        edge_ref[...] = ed
        ss = jnp.sum(ed * ed, axis=1, keepdims=True)
        ne = jnp.maximum(jnp.sqrt(ss), 1e-8)
        ex_ref[...] = jnp.concatenate(
            [ed / ne,
             jnp.broadcast_to(ne, (bm, L)),
             jnp.zeros((bm, DH - L), f32)],
            axis=1,
        )

    return pl.pallas_call(
        body,
        grid=(M // bm,),
        in_specs=[
            pl.BlockSpec((1, 1), lambda i: (0, 0)),
            pl.BlockSpec((1, 1), lambda i: (0, 0)),
            pl.BlockSpec((NC, bm, DH), lambda i: (0, i, 0)),
            pl.BlockSpec((NC, bm, DH), lambda i: (0, i, 0)),
            pl.BlockSpec((NC, bm, DH), lambda i: (0, i, 0)),
            pl.BlockSpec((bm, D), lambda i: (i, 0)),
            pl.BlockSpec((D, D), lambda i: (0, 0)),
        ],
        out_specs=[
            pl.BlockSpec((bm, D), lambda i: (i, 0)),
            pl.BlockSpec((bm, DX), lambda i: (i, 0)),
        ],
        out_shape=[
            jax.ShapeDtypeStruct((M, D), f32),
            jax.ShapeDtypeStruct((M, DX), f32),
        ],
    )(alpha2, beta2, eslo, eshi, cnte, e0, W_edge)


def _node_dense(alpha2, beta2, nslo, nshi, cntv, v0, W_node, bn):
    N, D = v0.shape

    def body(a_ref, b_ref, lo_ref, hi_ref, cv_ref, v0_ref, w_ref, node_ref):
        a = a_ref[0, 0]
        bt = b_ref[0, 0]
        ns = jnp.concatenate(
            [lo_ref[0] + lo_ref[1], hi_ref[0] + hi_ref[1]], axis=1
        )
        cnt = jnp.maximum(cv_ref[0, :, 0:1] + cv_ref[1, :, 0:1], 1.0)
        mix = (1.0 - a) * (ns / cnt) + a * v0_ref[...]
        mm = lax.dot_general(mix, w_ref[...], (((1,), (1,)), ((), ())),
                             preferred_element_type=f32)
        node_ref[...] = bt * mm + (1.0 - bt) * mix

    return pl.pallas_call(
        body,
        grid=(N // bn,),
        in_specs=[
            pl.BlockSpec((1, 1), lambda i: (0, 0)),
            pl.BlockSpec((1, 1), lambda i: (0, 0)),
            pl.BlockSpec((NC, bn, DH), lambda i: (0, i, 0)),
            pl.BlockSpec((NC, bn, DH), lambda i: (0, i, 0)),
            pl.BlockSpec((NC, bn, DH), lambda i: (0, i, 0)),
            pl.BlockSpec((bn, D), lambda i: (i, 0)),
            pl.BlockSpec((D, D), lambda i: (0, 0)),
        ],
        out_specs=pl.BlockSpec((bn, D), lambda i: (i, 0)),
        out_shape=jax.ShapeDtypeStruct((N, D), f32),
    )(alpha2, beta2, nslo, nshi, cntv, v0, W_node)


def kernel(v, e, v0, e0, alpha, beta, vidx, eidx, W_node, W_edge):
    N, D = v.shape
    M = e0.shape[0]
    E = vidx.shape[0]
    vidx = vidx.astype(jnp.int32)
    eidx = eidx.astype(jnp.int32)
    alpha2 = jnp.reshape(alpha, (1, 1)).astype(f32)
    beta2 = jnp.reshape(beta, (1, 1)).astype(f32)

    zn = jnp.zeros((N, DH), f32)

    eslo, eshi, cnte, cntv = _build_phase_a(N, M, E, 64)(
        v[:, :DH], v[:, DH:], vidx, eidx, zn, jnp.ones((64, DH), f32),
    )

    vhat = _vhat(v, 1000)
    edge, ex = _edge_dense(alpha2, beta2, eslo, eshi, cnte, e0, W_edge, 200)

    (sims,) = _build_sims(N, M, E, D, 64)(vhat, ex, vidx, eidx)

    nslo, nshi = _build_scatter_halves(N, M, E, 80)(
        ex[:, :DH], ex[:, DH:D], sims, vidx, eidx, zn,
    )

    node = _node_dense(alpha2, beta2, nslo, nshi, cntv, v0, W_node, 200)
    return (node, edge)


# speedup vs baseline: 2.2625x; 1.0686x over previous
import functools

import jax
import jax.numpy as jnp
from jax import lax
from jax.experimental import pallas as pl
from jax.experimental.pallas import tpu as pltpu
from jax.experimental.pallas import tpu_sc as plsc

NC = 2
NS = 16
L = 16
NW = NC * NS
DH = 128

f32 = jnp.float32


def _sc_mesh():
    return plsc.VectorSubcoreMesh(
        core_axis_name="c", subcore_axis_name="s", num_cores=NC, num_subcores=NS
    )


def _splits(total):
    main = (total // NS) // 8 * 8
    return main, total - main * NS


def _chunk_counts(E, ch):
    assert E % ch == 0
    nchunks = E // ch
    return nchunks // NW, nchunks % NW


def _zero_rows(zsrc, dst, s, row_main, row_tail):
    r0 = s * row_main
    pltpu.sync_copy(zsrc.at[pl.ds(r0, row_main)], dst.at[pl.ds(r0, row_main)])
    if row_tail:
        @pl.when(s == NS - 1)
        def _():
            b = row_main * NS
            pltpu.sync_copy(zsrc.at[pl.ds(b, row_tail)], dst.at[pl.ds(b, row_tail)])


def _dump_rows(src, dst, s, row_main, row_tail):
    r0 = s * row_main
    pltpu.sync_copy(src.at[pl.ds(r0, row_main)], dst.at[pl.ds(r0, row_main)])
    if row_tail:
        @pl.when(s == NS - 1)
        def _():
            b = row_main * NS
            pltpu.sync_copy(src.at[pl.ds(b, row_tail)], dst.at[pl.ds(b, row_tail)])


def _pipelined_chunks(nt, ntmax, issue, work):
    issue(0, 0)
    issue(1, 1)

    def pair(p, carry):
        for sl in (0, 1):
            t = 2 * p + sl

            @pl.when(t < nt)
            def _():
                work(t, sl)

            @pl.when(t + 2 < nt)
            def _():
                issue(t + 2, sl)
        return carry

    lax.fori_loop(0, (ntmax + 1) // 2, pair, 0)


def _build_phase_a(N, M, E, ch):
    ntbase, ntrem = _chunk_counts(E, ch)
    ntmax = ntbase + (1 if ntrem else 0)
    rm, rm_tail = _splits(M)
    rn, rn_tail = _splits(N)

    @functools.partial(
        pl.kernel,
        mesh=_sc_mesh(),
        out_type=[
            jax.ShapeDtypeStruct((NC, M, DH), f32),
            jax.ShapeDtypeStruct((NC, M, DH), f32),
            jax.ShapeDtypeStruct((NC, M, DH), f32),
            jax.ShapeDtypeStruct((NC, N, DH), f32),
        ],
        scratch_types=[
            pltpu.VMEM((ch,), jnp.int32),
            pltpu.VMEM((ch,), jnp.int32),
            pltpu.VMEM((ch,), jnp.int32),
            pltpu.VMEM((ch,), jnp.int32),
            pltpu.VMEM((ch, DH), f32),
            pltpu.VMEM((ch, DH), f32),
            pltpu.VMEM((ch, DH), f32),
            pltpu.VMEM_SHARED((N, DH), f32),
            pltpu.SemaphoreType.DMA,
            pltpu.SemaphoreType.DMA,
        ],
    )
    def phase_a(vlo_h, vhi_h, vidx_h, eidx_h, zn_h, ones_h,
                eslo_o, eshi_o, cnte_o, cntv_o,
                vi0, vi1, ei0, ei1, rows0, rows1, ones_v, acc, sem0, sem1):
        c = lax.axis_index("c")
        s = lax.axis_index("s")
        wid = s * NC + c
        nt = ntbase + jnp.where(wid < ntrem, 1, 0)
        pltpu.sync_copy(ones_h, ones_v)
        vi = (vi0, vi1)
        ei = (ei0, ei1)
        rows = (rows0, rows1)
        sems = (sem0, sem1)

        def gather_stage(table_h, out_ref, rows_main, rows_tail):
            _zero_rows(zn_h, acc, s, rn, rn_tail)
            plsc.subcore_barrier()

            def issue(t, sl):
                off = pl.multiple_of((wid + t * NW) * ch, 8)
                pltpu.sync_copy(vidx_h.at[pl.ds(off, ch)], vi[sl])
                pltpu.sync_copy(eidx_h.at[pl.ds(off, ch)], ei[sl])
                pltpu.async_copy(table_h.at[vi[sl]], rows[sl], sems[sl])

            def work(t, sl):
                pltpu.make_async_copy(table_h.at[vi[sl]], rows[sl], sems[sl]).wait()
                pltpu.sync_copy(rows[sl], acc.at[ei[sl]], add=True)

            _pipelined_chunks(nt, ntmax, issue, work)
            plsc.subcore_barrier()
            _dump_rows(acc, out_ref.at[c], s, rows_main, rows_tail)
            plsc.subcore_barrier()

        def count_stage(out_ref, rows_main, rows_tail, by_vidx):
            _zero_rows(zn_h, acc, s, rn, rn_tail)
            plsc.subcore_barrier()
            idx_h = vidx_h if by_vidx else eidx_h

            def issue(t, sl):
                off = pl.multiple_of((wid + t * NW) * ch, 8)
                pltpu.async_copy(idx_h.at[pl.ds(off, ch)], vi[sl], sems[sl])

            def work(t, sl):
                off = pl.multiple_of((wid + t * NW) * ch, 8)
                pltpu.make_async_copy(idx_h.at[pl.ds(off, ch)], vi[sl], sems[sl]).wait()
                pltpu.sync_copy(ones_v, acc.at[vi[sl]], add=True)

            _pipelined_chunks(nt, ntmax, issue, work)
            plsc.subcore_barrier()
            _dump_rows(acc, out_ref.at[c], s, rows_main, rows_tail)
            plsc.subcore_barrier()

        gather_stage(vlo_h, eslo_o, rm, rm_tail)
        gather_stage(vhi_h, eshi_o, rm, rm_tail)
        count_stage(cnte_o, rm, rm_tail, by_vidx=False)
        count_stage(cntv_o, rn, rn_tail, by_vidx=True)

    return phase_a


def _build_sims(N, M, E, D, ch):
    ntbase, ntrem = _chunk_counts(E, ch)
    ntmax = ntbase + (1 if ntrem else 0)
    KD = D // L

    @functools.partial(
        pl.kernel,
        mesh=_sc_mesh(),
        out_type=[jax.ShapeDtypeStruct((E, L), f32)],
        scratch_types=[
            pltpu.VMEM((ch,), jnp.int32),
            pltpu.VMEM((ch,), jnp.int32),
            pltpu.VMEM((ch,), jnp.int32),
            pltpu.VMEM((ch,), jnp.int32),
            pltpu.VMEM((ch, D), f32),
            pltpu.VMEM((ch, D), f32),
            pltpu.VMEM((ch, D), f32),
            pltpu.VMEM((ch, D), f32),
            pltpu.VMEM((ch, L), f32),
            pltpu.VMEM((ch, 2 * L), f32),
            pltpu.SemaphoreType.DMA,
            pltpu.SemaphoreType.DMA,
            pltpu.SemaphoreType.DMA,
            pltpu.SemaphoreType.DMA,
        ],
    )
    def sims_k(vhat_h, ehat_h, vidx_h, eidx_h,
               sims_o,
               vi0, vi1, ei0, ei1, vrows0, vrows1, erows0, erows1,
               sims_v, tmp_v, semv0, semv1, seme0, seme1):
        c = lax.axis_index("c")
        s = lax.axis_index("s")
        wid = s * NC + c
        nt = ntbase + jnp.where(wid < ntrem, 1, 0)
        vi = (vi0, vi1)
        ei = (ei0, ei1)
        vrows = (vrows0, vrows1)
        erows = (erows0, erows1)
        semv = (semv0, semv1)
        seme = (seme0, seme1)

        def issue(t, sl):
            off = pl.multiple_of((wid + t * NW) * ch, 8)
            pltpu.sync_copy(vidx_h.at[pl.ds(off, ch)], vi[sl])
            pltpu.sync_copy(eidx_h.at[pl.ds(off, ch)], ei[sl])
            pltpu.async_copy(vhat_h.at[vi[sl]], vrows[sl], semv[sl])
            pltpu.async_copy(ehat_h.at[ei[sl]], erows[sl], seme[sl])

        def work(t, sl):
            off = pl.multiple_of((wid + t * NW) * ch, 8)
            pltpu.make_async_copy(vhat_h.at[vi[sl]], vrows[sl], semv[sl]).wait()
            pltpu.make_async_copy(ehat_h.at[ei[sl]], erows[sl], seme[sl]).wait()
            vr = vrows[sl]
            er = erows[sl]

            def one(i):
                accs = [
                    vr[i, pl.ds(k * L, L)] * er[i, pl.ds(k * L, L)]
                    for k in range(4)
                ]
                for k in range(4, KD):
                    accs[k % 4] = accs[k % 4] + (
                        vr[i, pl.ds(k * L, L)] * er[i, pl.ds(k * L, L)]
                    )
                d = (accs[0] + accs[1]) + (accs[2] + accs[3])
                for sh in (8, 4, 2, 1):
                    tmp_v[i, pl.ds(0, L)] = d
                    tmp_v[i, pl.ds(L, L)] = d
                    d = d + tmp_v[i, pl.ds(sh, L)]
                sims_v[i] = 1.0 / (1.0 + jnp.exp(-d))

            def ibody(i2, carry2):
                one(2 * i2)
                one(2 * i2 + 1)
                return carry2

            lax.fori_loop(0, ch // 2, ibody, 0)
            pltpu.sync_copy(sims_v, sims_o.at[pl.ds(off, ch)])

        _pipelined_chunks(nt, ntmax, issue, work)

    return sims_k


def _build_scatter_halves(N, M, E, ch):
    ntbase, ntrem = _chunk_counts(E, ch)
    ntmax = ntbase + (1 if ntrem else 0)
    rn, rn_tail = _splits(N)
    KH = DH // L

    @functools.partial(
        pl.kernel,
        mesh=_sc_mesh(),
        out_type=[
            jax.ShapeDtypeStruct((NC, N, DH), f32),
            jax.ShapeDtypeStruct((NC, N, DH), f32),
        ],
        scratch_types=[
            pltpu.VMEM((ch,), jnp.int32),
            pltpu.VMEM((ch,), jnp.int32),
            pltpu.VMEM((ch,), jnp.int32),
            pltpu.VMEM((ch,), jnp.int32),
            pltpu.VMEM((ch, DH), f32),
            pltpu.VMEM((ch, DH), f32),
            pltpu.VMEM((ch, L), f32),
            pltpu.VMEM((ch, L), f32),
            pltpu.VMEM_SHARED((N, DH), f32),
            pltpu.SemaphoreType.DMA,
            pltpu.SemaphoreType.DMA,
        ],
    )
    def scatter_k(el_h, eh_h, sims_h, vidx_h, eidx_h, zn_h,
                  nslo_o, nshi_o,
                  vi0, vi1, ei0, ei1, rows0, rows1, sims0, sims1,
                  acc, sem0, sem1):
        c = lax.axis_index("c")
        s = lax.axis_index("s")
        wid = s * NC + c
        nt = ntbase + jnp.where(wid < ntrem, 1, 0)
        vi = (vi0, vi1)
        ei = (ei0, ei1)
        rows = (rows0, rows1)
        sims = (sims0, sims1)
        sems = (sem0, sem1)

        def stage(table_h, out_ref):
            _zero_rows(zn_h, acc, s, rn, rn_tail)
            plsc.subcore_barrier()

            def issue(t, sl):
                off = pl.multiple_of((wid + t * NW) * ch, 8)
                pltpu.sync_copy(vidx_h.at[pl.ds(off, ch)], vi[sl])
                pltpu.sync_copy(eidx_h.at[pl.ds(off, ch)], ei[sl])
                pltpu.sync_copy(sims_h.at[pl.ds(off, ch)], sims[sl])
                pltpu.async_copy(table_h.at[ei[sl]], rows[sl], sems[sl])

            def work(t, sl):
                pltpu.make_async_copy(table_h.at[ei[sl]], rows[sl], sems[sl]).wait()
                rr = rows[sl]
                sv = sims[sl]

                def one(i):
                    sw = sv[i]
                    for k in range(KH):
                        rr[i, pl.ds(k * L, L)] = sw * rr[i, pl.ds(k * L, L)]

                def ibody(i2, carry2):
                    one(2 * i2)
                    one(2 * i2 + 1)
                    return carry2

                lax.fori_loop(0, ch // 2, ibody, 0)
                pltpu.sync_copy(rr, acc.at[vi[sl]], add=True)

            _pipelined_chunks(nt, ntmax, issue, work)
            plsc.subcore_barrier()
            _dump_rows(acc, out_ref.at[c], s, rn, rn_tail)
            plsc.subcore_barrier()

        stage(el_h, nslo_o)
        stage(eh_h, nshi_o)

    return scatter_k


def _vhat(x, bn):
    n, d = x.shape

    def body(x_ref, o_ref):
        xb = x_ref[...]
        ss = jnp.sum(xb * xb, axis=1, keepdims=True)
        o_ref[...] = xb / jnp.maximum(jnp.sqrt(ss), 1e-8)

    return pl.pallas_call(
        body,
        grid=(n // bn,),
        in_specs=[pl.BlockSpec((bn, d), lambda i: (i, 0))],
        out_specs=pl.BlockSpec((bn, d), lambda i: (i, 0)),
        out_shape=jax.ShapeDtypeStruct((n, d), f32),
    )(x)


def _edge_dense(alpha2, beta2, eslo, eshi, cnte, e0, W_edge, bm):
    M, D = e0.shape

    def body(a_ref, b_ref, lo_ref, hi_ref, ce_ref, e0_ref, w_ref,
             edge_ref, ehat_ref):
        a = a_ref[0, 0]
        bt = b_ref[0, 0]
        es = jnp.concatenate(
            [lo_ref[0] + lo_ref[1], hi_ref[0] + hi_ref[1]], axis=1
        )
        cnt = jnp.maximum(ce_ref[0, :, 0:1] + ce_ref[1, :, 0:1], 1.0)
        mix = (1.0 - a) * (es / cnt) + a * e0_ref[...]
        mm = lax.dot_general(mix, w_ref[...], (((1,), (1,)), ((), ())),
                             preferred_element_type=f32)
        ed = jnp.maximum(bt * mm + (1.0 - bt) * mix, 0.0)
        edge_ref[...] = ed
        ss = jnp.sum(ed * ed, axis=1, keepdims=True)
        ehat_ref[...] = ed / jnp.maximum(jnp.sqrt(ss), 1e-8)

    return pl.pallas_call(
        body,
        grid=(M // bm,),
        in_specs=[
            pl.BlockSpec((1, 1), lambda i: (0, 0)),
            pl.BlockSpec((1, 1), lambda i: (0, 0)),
            pl.BlockSpec((NC, bm, DH), lambda i: (0, i, 0)),
            pl.BlockSpec((NC, bm, DH), lambda i: (0, i, 0)),
            pl.BlockSpec((NC, bm, DH), lambda i: (0, i, 0)),
            pl.BlockSpec((bm, D), lambda i: (i, 0)),
            pl.BlockSpec((D, D), lambda i: (0, 0)),
        ],
        out_specs=[
            pl.BlockSpec((bm, D), lambda i: (i, 0)),
            pl.BlockSpec((bm, D), lambda i: (i, 0)),
        ],
        out_shape=[
            jax.ShapeDtypeStruct((M, D), f32),
            jax.ShapeDtypeStruct((M, D), f32),
        ],
    )(alpha2, beta2, eslo, eshi, cnte, e0, W_edge)


def _node_dense(alpha2, beta2, nslo, nshi, cntv, v0, W_node, bn):
    N, D = v0.shape

    def body(a_ref, b_ref, lo_ref, hi_ref, cv_ref, v0_ref, w_ref, node_ref):
        a = a_ref[0, 0]
        bt = b_ref[0, 0]
        ns = jnp.concatenate(
            [lo_ref[0] + lo_ref[1], hi_ref[0] + hi_ref[1]], axis=1
        )
        cnt = jnp.maximum(cv_ref[0, :, 0:1] + cv_ref[1, :, 0:1], 1.0)
        mix = (1.0 - a) * (ns / cnt) + a * v0_ref[...]
        mm = lax.dot_general(mix, w_ref[...], (((1,), (1,)), ((), ())),
                             preferred_element_type=f32)
        node_ref[...] = bt * mm + (1.0 - bt) * mix

    return pl.pallas_call(
        body,
        grid=(N // bn,),
        in_specs=[
            pl.BlockSpec((1, 1), lambda i: (0, 0)),
            pl.BlockSpec((1, 1), lambda i: (0, 0)),
            pl.BlockSpec((NC, bn, DH), lambda i: (0, i, 0)),
            pl.BlockSpec((NC, bn, DH), lambda i: (0, i, 0)),
            pl.BlockSpec((NC, bn, DH), lambda i: (0, i, 0)),
            pl.BlockSpec((bn, D), lambda i: (i, 0)),
            pl.BlockSpec((D, D), lambda i: (0, 0)),
        ],
        out_specs=pl.BlockSpec((bn, D), lambda i: (i, 0)),
        out_shape=jax.ShapeDtypeStruct((N, D), f32),
    )(alpha2, beta2, nslo, nshi, cntv, v0, W_node)


def kernel(v, e, v0, e0, alpha, beta, vidx, eidx, W_node, W_edge):
    N, D = v.shape
    M = e0.shape[0]
    E = vidx.shape[0]
    vidx = vidx.astype(jnp.int32)
    eidx = eidx.astype(jnp.int32)
    alpha2 = jnp.reshape(alpha, (1, 1)).astype(f32)
    beta2 = jnp.reshape(beta, (1, 1)).astype(f32)

    zn = jnp.zeros((N, DH), f32)

    eslo, eshi, cnte, cntv = _build_phase_a(N, M, E, 64)(
        v[:, :DH], v[:, DH:], vidx, eidx, zn, jnp.ones((64, DH), f32),
    )

    vhat = _vhat(v, 1000)
    edge, ehat = _edge_dense(alpha2, beta2, eslo, eshi, cnte, e0, W_edge, 200)

    (sims,) = _build_sims(N, M, E, D, 80)(vhat, ehat, vidx, eidx)

    nslo, nshi = _build_scatter_halves(N, M, E, 80)(
        edge[:, :DH], edge[:, DH:], sims, vidx, eidx, zn,
    )

    node = _node_dense(alpha2, beta2, nslo, nshi, cntv, v0, W_node, 200)
    return (node, edge)


# speedup vs baseline: 2.3281x; 1.0290x over previous
import functools

import jax
import jax.numpy as jnp
from jax import lax
from jax.experimental import pallas as pl
from jax.experimental.pallas import tpu as pltpu
from jax.experimental.pallas import tpu_sc as plsc

NC = 2
NS = 16
L = 16
NW = NC * NS
DH = 128

f32 = jnp.float32


def _sc_mesh():
    return plsc.VectorSubcoreMesh(
        core_axis_name="c", subcore_axis_name="s", num_cores=NC, num_subcores=NS
    )


def _splits(total):
    main = (total // NS) // 8 * 8
    return main, total - main * NS


def _chunk_counts(E, ch):
    assert E % ch == 0
    nchunks = E // ch
    return nchunks // NW, nchunks % NW


def _zero_rows(zsrc, dst, s, row_main, row_tail):
    r0 = s * row_main
    pltpu.sync_copy(zsrc.at[pl.ds(r0, row_main)], dst.at[pl.ds(r0, row_main)])
    if row_tail:
        @pl.when(s == NS - 1)
        def _():
            b = row_main * NS
            pltpu.sync_copy(zsrc.at[pl.ds(b, row_tail)], dst.at[pl.ds(b, row_tail)])


def _dump_rows(src, dst, s, row_main, row_tail):
    r0 = s * row_main
    pltpu.sync_copy(src.at[pl.ds(r0, row_main)], dst.at[pl.ds(r0, row_main)])
    if row_tail:
        @pl.when(s == NS - 1)
        def _():
            b = row_main * NS
            pltpu.sync_copy(src.at[pl.ds(b, row_tail)], dst.at[pl.ds(b, row_tail)])


def _pipelined_chunks(nt, ntmax, issue, work):
    issue(0, 0)
    issue(1, 1)

    def pair(p, carry):
        for sl in (0, 1):
            t = 2 * p + sl

            @pl.when(t < nt)
            def _():
                work(t, sl)

            @pl.when(t + 2 < nt)
            def _():
                issue(t + 2, sl)
        return carry

    lax.fori_loop(0, (ntmax + 1) // 2, pair, 0)


def _build_phase_a(N, M, E, ch):
    ntbase, ntrem = _chunk_counts(E, ch)
    ntmax = ntbase + (1 if ntrem else 0)
    rm, rm_tail = _splits(M)
    rn, rn_tail = _splits(N)

    @functools.partial(
        pl.kernel,
        mesh=_sc_mesh(),
        out_type=[
            jax.ShapeDtypeStruct((NC, M, DH), f32),
            jax.ShapeDtypeStruct((NC, M, DH), f32),
            jax.ShapeDtypeStruct((NC, M, DH), f32),
            jax.ShapeDtypeStruct((NC, N, DH), f32),
        ],
        scratch_types=[
            pltpu.VMEM((ch,), jnp.int32),
            pltpu.VMEM((ch,), jnp.int32),
            pltpu.VMEM((ch,), jnp.int32),
            pltpu.VMEM((ch,), jnp.int32),
            pltpu.VMEM((ch, DH), f32),
            pltpu.VMEM((ch, DH), f32),
            pltpu.VMEM((ch, DH), f32),
            pltpu.VMEM_SHARED((N, DH), f32),
            pltpu.SemaphoreType.DMA,
            pltpu.SemaphoreType.DMA,
        ],
    )
    def phase_a(vlo_h, vhi_h, vidx_h, eidx_h, zn_h, ones_h,
                eslo_o, eshi_o, cnte_o, cntv_o,
                vi0, vi1, ei0, ei1, rows0, rows1, ones_v, acc, sem0, sem1):
        c = lax.axis_index("c")
        s = lax.axis_index("s")
        wid = s * NC + c
        nt = ntbase + jnp.where(wid < ntrem, 1, 0)
        pltpu.sync_copy(ones_h, ones_v)
        vi = (vi0, vi1)
        ei = (ei0, ei1)
        rows = (rows0, rows1)
        sems = (sem0, sem1)

        def gather_stage(table_h, out_ref, rows_main, rows_tail):
            _zero_rows(zn_h, acc, s, rn, rn_tail)
            plsc.subcore_barrier()

            def issue(t, sl):
                off = pl.multiple_of((wid + t * NW) * ch, 8)
                pltpu.sync_copy(vidx_h.at[pl.ds(off, ch)], vi[sl])
                pltpu.sync_copy(eidx_h.at[pl.ds(off, ch)], ei[sl])
                pltpu.async_copy(table_h.at[vi[sl]], rows[sl], sems[sl])

            def work(t, sl):
                pltpu.make_async_copy(table_h.at[vi[sl]], rows[sl], sems[sl]).wait()
                pltpu.sync_copy(rows[sl], acc.at[ei[sl]], add=True)

            _pipelined_chunks(nt, ntmax, issue, work)
            plsc.subcore_barrier()
            _dump_rows(acc, out_ref.at[c], s, rows_main, rows_tail)
            plsc.subcore_barrier()

        def count_stage(out_ref, rows_main, rows_tail, by_vidx):
            _zero_rows(zn_h, acc, s, rn, rn_tail)
            plsc.subcore_barrier()
            idx_h = vidx_h if by_vidx else eidx_h

            def issue(t, sl):
                off = pl.multiple_of((wid + t * NW) * ch, 8)
                pltpu.async_copy(idx_h.at[pl.ds(off, ch)], vi[sl], sems[sl])

            def work(t, sl):
                off = pl.multiple_of((wid + t * NW) * ch, 8)
                pltpu.make_async_copy(idx_h.at[pl.ds(off, ch)], vi[sl], sems[sl]).wait()
                pltpu.sync_copy(ones_v, acc.at[vi[sl]], add=True)

            _pipelined_chunks(nt, ntmax, issue, work)
            plsc.subcore_barrier()
            _dump_rows(acc, out_ref.at[c], s, rows_main, rows_tail)
            plsc.subcore_barrier()

        gather_stage(vlo_h, eslo_o, rm, rm_tail)
        gather_stage(vhi_h, eshi_o, rm, rm_tail)
        count_stage(cnte_o, rm, rm_tail, by_vidx=False)
        count_stage(cntv_o, rn, rn_tail, by_vidx=True)

    return phase_a


def _build_sims(N, M, E, D, ch):
    ntbase, ntrem = _chunk_counts(E, ch)
    ntmax = ntbase + (1 if ntrem else 0)
    KD = D // L

    @functools.partial(
        pl.kernel,
        mesh=_sc_mesh(),
        out_type=[jax.ShapeDtypeStruct((E, L), f32)],
        scratch_types=[
            pltpu.VMEM((ch,), jnp.int32),
            pltpu.VMEM((ch,), jnp.int32),
            pltpu.VMEM((ch,), jnp.int32),
            pltpu.VMEM((ch,), jnp.int32),
            pltpu.VMEM((ch, D), f32),
            pltpu.VMEM((ch, D), f32),
            pltpu.VMEM((ch, D), f32),
            pltpu.VMEM((ch, D), f32),
            pltpu.VMEM((ch, L), f32),
            pltpu.VMEM((ch, 2 * L), f32),
            pltpu.SemaphoreType.DMA,
            pltpu.SemaphoreType.DMA,
            pltpu.SemaphoreType.DMA,
            pltpu.SemaphoreType.DMA,
        ],
    )
    def sims_k(vhat_h, ehat_h, vidx_h, eidx_h,
               sims_o,
               vi0, vi1, ei0, ei1, vrows0, vrows1, erows0, erows1,
               sims_v, tmp_v, semv0, semv1, seme0, seme1):
        c = lax.axis_index("c")
        s = lax.axis_index("s")
        wid = s * NC + c
        nt = ntbase + jnp.where(wid < ntrem, 1, 0)
        vi = (vi0, vi1)
        ei = (ei0, ei1)
        vrows = (vrows0, vrows1)
        erows = (erows0, erows1)
        semv = (semv0, semv1)
        seme = (seme0, seme1)

        def issue(t, sl):
            off = pl.multiple_of((wid + t * NW) * ch, 8)
            pltpu.sync_copy(vidx_h.at[pl.ds(off, ch)], vi[sl])
            pltpu.sync_copy(eidx_h.at[pl.ds(off, ch)], ei[sl])
            pltpu.async_copy(vhat_h.at[vi[sl]], vrows[sl], semv[sl])
            pltpu.async_copy(ehat_h.at[ei[sl]], erows[sl], seme[sl])

        def work(t, sl):
            off = pl.multiple_of((wid + t * NW) * ch, 8)
            pltpu.make_async_copy(vhat_h.at[vi[sl]], vrows[sl], semv[sl]).wait()
            pltpu.make_async_copy(ehat_h.at[ei[sl]], erows[sl], seme[sl]).wait()
            vr = vrows[sl]
            er = erows[sl]

            def one(i):
                accs = [
                    vr[i, pl.ds(k * L, L)] * er[i, pl.ds(k * L, L)]
                    for k in range(4)
                ]
                for k in range(4, KD):
                    accs[k % 4] = accs[k % 4] + (
                        vr[i, pl.ds(k * L, L)] * er[i, pl.ds(k * L, L)]
                    )
                d = (accs[0] + accs[1]) + (accs[2] + accs[3])
                for sh in (8, 4, 2, 1):
                    tmp_v[i, pl.ds(0, L)] = d
                    tmp_v[i, pl.ds(L, L)] = d
                    d = d + tmp_v[i, pl.ds(sh, L)]
                sims_v[i] = 1.0 / (1.0 + jnp.exp(-d))

            def ibody(i4, carry2):
                one(4 * i4)
                one(4 * i4 + 1)
                one(4 * i4 + 2)
                one(4 * i4 + 3)
                return carry2

            lax.fori_loop(0, ch // 4, ibody, 0)
            pltpu.sync_copy(sims_v, sims_o.at[pl.ds(off, ch)])

        _pipelined_chunks(nt, ntmax, issue, work)

    return sims_k


def _build_scatter_halves(N, M, E, ch):
    ntbase, ntrem = _chunk_counts(E, ch)
    ntmax = ntbase + (1 if ntrem else 0)
    rn, rn_tail = _splits(N)
    KH = DH // L

    @functools.partial(
        pl.kernel,
        mesh=_sc_mesh(),
        out_type=[
            jax.ShapeDtypeStruct((NC, N, DH), f32),
            jax.ShapeDtypeStruct((NC, N, DH), f32),
        ],
        scratch_types=[
            pltpu.VMEM((ch,), jnp.int32),
            pltpu.VMEM((ch,), jnp.int32),
            pltpu.VMEM((ch,), jnp.int32),
            pltpu.VMEM((ch,), jnp.int32),
            pltpu.VMEM((ch, DH), f32),
            pltpu.VMEM((ch, DH), f32),
            pltpu.VMEM((ch, L), f32),
            pltpu.VMEM((ch, L), f32),
            pltpu.VMEM_SHARED((N, DH), f32),
            pltpu.SemaphoreType.DMA,
            pltpu.SemaphoreType.DMA,
        ],
    )
    def scatter_k(el_h, eh_h, sims_h, vidx_h, eidx_h, zn_h,
                  nslo_o, nshi_o,
                  vi0, vi1, ei0, ei1, rows0, rows1, sims0, sims1,
                  acc, sem0, sem1):
        c = lax.axis_index("c")
        s = lax.axis_index("s")
        wid = s * NC + c
        nt = ntbase + jnp.where(wid < ntrem, 1, 0)
        vi = (vi0, vi1)
        ei = (ei0, ei1)
        rows = (rows0, rows1)
        sims = (sims0, sims1)
        sems = (sem0, sem1)

        def stage(table_h, out_ref):
            _zero_rows(zn_h, acc, s, rn, rn_tail)
            plsc.subcore_barrier()

            def issue(t, sl):
                off = pl.multiple_of((wid + t * NW) * ch, 8)
                pltpu.sync_copy(vidx_h.at[pl.ds(off, ch)], vi[sl])
                pltpu.sync_copy(eidx_h.at[pl.ds(off, ch)], ei[sl])
                pltpu.sync_copy(sims_h.at[pl.ds(off, ch)], sims[sl])
                pltpu.async_copy(table_h.at[ei[sl]], rows[sl], sems[sl])

            def work(t, sl):
                pltpu.make_async_copy(table_h.at[ei[sl]], rows[sl], sems[sl]).wait()
                rr = rows[sl]
                sv = sims[sl]

                def one(i):
                    sw = sv[i]
                    for k in range(KH):
                        rr[i, pl.ds(k * L, L)] = sw * rr[i, pl.ds(k * L, L)]

                def ibody(i2, carry2):
                    one(2 * i2)
                    one(2 * i2 + 1)
                    return carry2

                lax.fori_loop(0, ch // 2, ibody, 0)
                pltpu.sync_copy(rr, acc.at[vi[sl]], add=True)

            _pipelined_chunks(nt, ntmax, issue, work)
            plsc.subcore_barrier()
            _dump_rows(acc, out_ref.at[c], s, rn, rn_tail)
            plsc.subcore_barrier()

        stage(el_h, nslo_o)
        stage(eh_h, nshi_o)

    return scatter_k


def _vhat(x, bn):
    n, d = x.shape

    def body(x_ref, o_ref):
        xb = x_ref[...]
        ss = jnp.sum(xb * xb, axis=1, keepdims=True)
        o_ref[...] = xb / jnp.maximum(jnp.sqrt(ss), 1e-8)

    return pl.pallas_call(
        body,
        grid=(n // bn,),
        in_specs=[pl.BlockSpec((bn, d), lambda i: (i, 0))],
        out_specs=pl.BlockSpec((bn, d), lambda i: (i, 0)),
        out_shape=jax.ShapeDtypeStruct((n, d), f32),
    )(x)


def _edge_dense(alpha2, beta2, eslo, eshi, cnte, e0, W_edge, bm):
    M, D = e0.shape

    def body(a_ref, b_ref, lo_ref, hi_ref, ce_ref, e0_ref, w_ref,
             edge_ref, ehat_ref):
        a = a_ref[0, 0]
        bt = b_ref[0, 0]
        es = jnp.concatenate(
            [lo_ref[0] + lo_ref[1], hi_ref[0] + hi_ref[1]], axis=1
        )
        cnt = jnp.maximum(ce_ref[0, :, 0:1] + ce_ref[1, :, 0:1], 1.0)
        mix = (1.0 - a) * (es / cnt) + a * e0_ref[...]
        mm = lax.dot_general(mix, w_ref[...], (((1,), (1,)), ((), ())),
                             preferred_element_type=f32)
        ed = jnp.maximum(bt * mm + (1.0 - bt) * mix, 0.0)
        edge_ref[...] = ed
        ss = jnp.sum(ed * ed, axis=1, keepdims=True)
        ehat_ref[...] = ed / jnp.maximum(jnp.sqrt(ss), 1e-8)

    return pl.pallas_call(
        body,
        grid=(M // bm,),
        in_specs=[
            pl.BlockSpec((1, 1), lambda i: (0, 0)),
            pl.BlockSpec((1, 1), lambda i: (0, 0)),
            pl.BlockSpec((NC, bm, DH), lambda i: (0, i, 0)),
            pl.BlockSpec((NC, bm, DH), lambda i: (0, i, 0)),
            pl.BlockSpec((NC, bm, DH), lambda i: (0, i, 0)),
            pl.BlockSpec((bm, D), lambda i: (i, 0)),
            pl.BlockSpec((D, D), lambda i: (0, 0)),
        ],
        out_specs=[
            pl.BlockSpec((bm, D), lambda i: (i, 0)),
            pl.BlockSpec((bm, D), lambda i: (i, 0)),
        ],
        out_shape=[
            jax.ShapeDtypeStruct((M, D), f32),
            jax.ShapeDtypeStruct((M, D), f32),
        ],
    )(alpha2, beta2, eslo, eshi, cnte, e0, W_edge)


def _node_dense(alpha2, beta2, nslo, nshi, cntv, v0, W_node, bn):
    N, D = v0.shape

    def body(a_ref, b_ref, lo_ref, hi_ref, cv_ref, v0_ref, w_ref, node_ref):
        a = a_ref[0, 0]
        bt = b_ref[0, 0]
        ns = jnp.concatenate(
            [lo_ref[0] + lo_ref[1], hi_ref[0] + hi_ref[1]], axis=1
        )
        cnt = jnp.maximum(cv_ref[0, :, 0:1] + cv_ref[1, :, 0:1], 1.0)
        mix = (1.0 - a) * (ns / cnt) + a * v0_ref[...]
        mm = lax.dot_general(mix, w_ref[...], (((1,), (1,)), ((), ())),
                             preferred_element_type=f32)
        node_ref[...] = bt * mm + (1.0 - bt) * mix

    return pl.pallas_call(
        body,
        grid=(N // bn,),
        in_specs=[
            pl.BlockSpec((1, 1), lambda i: (0, 0)),
            pl.BlockSpec((1, 1), lambda i: (0, 0)),
            pl.BlockSpec((NC, bn, DH), lambda i: (0, i, 0)),
            pl.BlockSpec((NC, bn, DH), lambda i: (0, i, 0)),
            pl.BlockSpec((NC, bn, DH), lambda i: (0, i, 0)),
            pl.BlockSpec((bn, D), lambda i: (i, 0)),
            pl.BlockSpec((D, D), lambda i: (0, 0)),
        ],
        out_specs=pl.BlockSpec((bn, D), lambda i: (i, 0)),
        out_shape=jax.ShapeDtypeStruct((N, D), f32),
    )(alpha2, beta2, nslo, nshi, cntv, v0, W_node)


def kernel(v, e, v0, e0, alpha, beta, vidx, eidx, W_node, W_edge):
    N, D = v.shape
    M = e0.shape[0]
    E = vidx.shape[0]
    vidx = vidx.astype(jnp.int32)
    eidx = eidx.astype(jnp.int32)
    alpha2 = jnp.reshape(alpha, (1, 1)).astype(f32)
    beta2 = jnp.reshape(beta, (1, 1)).astype(f32)

    zn = jnp.zeros((N, DH), f32)

    eslo, eshi, cnte, cntv = _build_phase_a(N, M, E, 80)(
        v[:, :DH], v[:, DH:], vidx, eidx, zn, jnp.ones((80, DH), f32),
    )

    vhat = _vhat(v, 1000)
    edge, ehat = _edge_dense(alpha2, beta2, eslo, eshi, cnte, e0, W_edge, 200)

    (sims,) = _build_sims(N, M, E, D, 80)(vhat, ehat, vidx, eidx)

    nslo, nshi = _build_scatter_halves(N, M, E, 80)(
        edge[:, :DH], edge[:, DH:], sims, vidx, eidx, zn,
    )

    node = _node_dense(alpha2, beta2, nslo, nshi, cntv, v0, W_node, 200)
    return (node, edge)
